# baseline (speedup 1.0000x reference)
import functools

import jax
import jax.numpy as jnp
from jax import lax
from jax.experimental import pallas as pl
from jax.experimental.pallas import tpu as pltpu
from jax.experimental.pallas import tpu_sc as plsc

N_NODES = 10000
N_EDGES = 160000
D = 256
HALF = 128

NC = 2
NS = 16
EB = 128
NB = 80
NBH = 40
EPT = EB * NB
E_PAD = EPT * NS
ACC_N = 10240
DUMMY = 10016
RPT = ACC_N // NS

BR = 2000
G = N_NODES // BR


def _sc_sum_body(x_cat, src4, dst3, zacc, sum_out,
                 src_r, dst_v, db0, db1, rows0, rows1, acc_sh, sem0, sem1):
    cid = lax.axis_index("c")
    sid = lax.axis_index("s")

    zoff = sid * RPT
    pltpu.sync_copy(zacc.at[pl.ds(zoff, RPT)], acc_sh.at[pl.ds(zoff, RPT)])
    pltpu.sync_copy(dst3.at[sid], dst_v)
    plsc.subcore_barrier()

    def step(j, carry):
        pltpu.sync_copy(src4.at[cid, sid, j], src_r.at[0])
        pltpu.async_copy(x_cat.at[src_r.at[0]], rows0, sem0).wait()
        pltpu.sync_copy(rows0, acc_sh.at[dst_v.at[j]], add=True)
        return carry

    lax.fori_loop(0, NB, step, 0)
    plsc.subcore_barrier()

    soff = cid * ACC_N + sid * RPT
    pltpu.sync_copy(acc_sh.at[pl.ds(sid * RPT, RPT)],
                    sum_out.at[pl.ds(soff, RPT)])


def _sc_cnt_body(dst3, ones_h, zacc, cnt_out,
                 dst_v, db, ones_v, acc_sh):
    cid = lax.axis_index("c")
    sid = lax.axis_index("s")

    zoff = sid * RPT
    pltpu.sync_copy(zacc.at[pl.ds(zoff, RPT)], acc_sh.at[pl.ds(zoff, RPT)])
    pltpu.sync_copy(ones_h, ones_v)
    pltpu.sync_copy(dst3.at[sid, pl.ds(cid * NBH, NBH)], dst_v)
    plsc.subcore_barrier()

    def step(j, carry):
        pltpu.sync_copy(ones_v, acc_sh.at[dst_v.at[j]], add=True)
        return carry

    lax.fori_loop(0, NBH, step, 0)
    plsc.subcore_barrier()

    soff = cid * ACC_N + sid * RPT
    pltpu.sync_copy(acc_sh.at[pl.ds(sid * RPT, RPT)],
                    cnt_out.at[pl.ds(soff, RPT)])


def _make_sc_sum():
    return functools.partial(
        pl.kernel,
        out_type=[jax.ShapeDtypeStruct((NC * ACC_N, HALF), jnp.float32)],
        mesh=plsc.VectorSubcoreMesh(core_axis_name="c", subcore_axis_name="s",
                                    num_cores=NC, num_subcores=NS),
        scratch_types=[
            pltpu.VMEM((2, EB), jnp.int32),
            pltpu.VMEM((NB, EB), jnp.int32),
            pltpu.VMEM((EB,), jnp.int32),
            pltpu.VMEM((EB,), jnp.int32),
            pltpu.VMEM((EB, HALF), jnp.float32),
            pltpu.VMEM((EB, HALF), jnp.float32),
            pltpu.VMEM_SHARED((ACC_N, HALF), jnp.float32),
            pltpu.SemaphoreType.DMA,
            pltpu.SemaphoreType.DMA,
        ],
    )(_sc_sum_body)


def _make_sc_cnt():
    return functools.partial(
        pl.kernel,
        out_type=[jax.ShapeDtypeStruct((NC * ACC_N, HALF), jnp.float32)],
        mesh=plsc.VectorSubcoreMesh(core_axis_name="c", subcore_axis_name="s",
                                    num_cores=NC, num_subcores=NS),
        scratch_types=[
            pltpu.VMEM((NBH, EB), jnp.int32),
            pltpu.VMEM((EB,), jnp.int32),
            pltpu.VMEM((EB, HALF), jnp.float32),
            pltpu.VMEM_SHARED((ACC_N, HALF), jnp.float32),
        ],
    )(_sc_cnt_body)


def _tc1_body(s0, s1, c0, c1, xb, wl, wr, bb, h_out, ps_out, pq_out):
    cnt = jnp.maximum(c0[:, :1] + c1[:, :1], 1.0)
    mean = jnp.concatenate([s0[...], s1[...]], axis=1) / cnt
    h = jnp.dot(mean, wl[...], preferred_element_type=jnp.float32)
    h = h + jnp.dot(xb[...], wr[...], preferred_element_type=jnp.float32)
    h = jnp.maximum(h + bb[...], 0.0)
    h_out[...] = h
    ps_out[...] = jnp.sum(h, axis=0, keepdims=True)[None]
    pq_out[...] = jnp.sum(h * h, axis=0, keepdims=True)[None]


def _tc2_body(h, ps, pq, gam, bet, y_out):
    inv_n = jnp.float32(1.0 / N_NODES)
    mu = jnp.sum(ps[...], axis=0) * inv_n
    ex2 = jnp.sum(pq[...], axis=0) * inv_n
    var = ex2 - mu * mu
    scale = lax.rsqrt(var + 1e-5) * gam[...]
    y_out[...] = (h[...] - mu) * scale + bet[...]


def kernel(x, edge_index, W_l, W_r, b, gamma, beta):
    src = edge_index[0]
    dst = edge_index[1]
    pad = E_PAD - N_EDGES
    srcp = jnp.concatenate([src, jnp.zeros((pad,), jnp.int32)])
    src4 = jnp.stack([srcp, srcp + N_NODES]).reshape(NC, NS, NB, EB)
    dst3 = jnp.concatenate(
        [dst, jnp.full((pad,), DUMMY, jnp.int32)]).reshape(NS, NB, EB)
    x_cat = jnp.concatenate([x[:, :HALF], x[:, HALF:]], axis=0)
    ones_h = jnp.ones((EB, HALF), jnp.float32)
    zacc = jnp.zeros((ACC_N, HALF), jnp.float32)

    (s,) = _make_sc_sum()(x_cat, src4, dst3, zacc)
    (c,) = _make_sc_cnt()(dst3, ones_h, zacc)
    s0 = s[:N_NODES]
    s1 = s[ACC_N:ACC_N + N_NODES]
    c0 = c[:N_NODES]
    c1 = c[ACC_N:ACC_N + N_NODES]

    h, ps, pq = pl.pallas_call(
        _tc1_body,
        grid=(G,),
        in_specs=[
            pl.BlockSpec((BR, HALF), lambda g: (g, 0)),
            pl.BlockSpec((BR, HALF), lambda g: (g, 0)),
            pl.BlockSpec((BR, HALF), lambda g: (g, 0)),
            pl.BlockSpec((BR, HALF), lambda g: (g, 0)),
            pl.BlockSpec((BR, D), lambda g: (g, 0)),
            pl.BlockSpec((D, D), lambda g: (0, 0)),
            pl.BlockSpec((D, D), lambda g: (0, 0)),
            pl.BlockSpec((1, D), lambda g: (0, 0)),
        ],
        out_specs=[
            pl.BlockSpec((BR, D), lambda g: (g, 0)),
            pl.BlockSpec((1, 1, D), lambda g: (g, 0, 0)),
            pl.BlockSpec((1, 1, D), lambda g: (g, 0, 0)),
        ],
        out_shape=[
            jax.ShapeDtypeStruct((N_NODES, D), jnp.float32),
            jax.ShapeDtypeStruct((G, 1, D), jnp.float32),
            jax.ShapeDtypeStruct((G, 1, D), jnp.float32),
        ],
    )(s0, s1, c0, c1, x, W_l.T, W_r.T, b.reshape(1, D))

    y = pl.pallas_call(
        _tc2_body,
        grid=(G,),
        in_specs=[
            pl.BlockSpec((BR, D), lambda g: (g, 0)),
            pl.BlockSpec((G, 1, D), lambda g: (0, 0, 0)),
            pl.BlockSpec((G, 1, D), lambda g: (0, 0, 0)),
            pl.BlockSpec((1, D), lambda g: (0, 0)),
            pl.BlockSpec((1, D), lambda g: (0, 0)),
        ],
        out_specs=pl.BlockSpec((BR, D), lambda g: (g, 0)),
        out_shape=jax.ShapeDtypeStruct((N_NODES, D), jnp.float32),
    )(h, ps, pq, gamma.reshape(1, D), beta.reshape(1, D))

    return y


# speedup vs baseline: 1.1828x; 1.1828x over previous
import functools

import jax
import jax.numpy as jnp
from jax import lax
from jax.experimental import pallas as pl
from jax.experimental.pallas import tpu as pltpu
from jax.experimental.pallas import tpu_sc as plsc

N_NODES = 10000
N_EDGES = 160000
D = 256
HALF = 128

NC = 2
NS = 16
EB = 128
NB = 80
NBH = 40
EPT = EB * NB
E_PAD = EPT * NS
ACC_N = 10240
DUMMY = 10016
RPT = ACC_N // NS

BR = 2000
G = N_NODES // BR


def _sc_sum_body(x_cat, src4, dst3, zacc, sum_out,
                 src_r, dst_v, db0, db1, rows0, rows1, acc_sh, sem0, sem1):
    cid = lax.axis_index("c")
    sid = lax.axis_index("s")

    zoff = sid * RPT
    pltpu.sync_copy(zacc.at[pl.ds(zoff, RPT)], acc_sh.at[pl.ds(zoff, RPT)])
    pltpu.sync_copy(dst3.at[sid], dst_v)
    plsc.subcore_barrier()

    rows = (rows0, rows1)
    sems = (sem0, sem1)
    pltpu.sync_copy(src4.at[cid, sid, 0], src_r.at[0])
    pltpu.sync_copy(src4.at[cid, sid, 1], src_r.at[1])
    pend = [pltpu.async_copy(x_cat.at[src_r.at[b]], rows[b], sems[b])
            for b in range(2)]
    for j in range(NB):
        b = j % 2
        pend[b].wait()
        pltpu.sync_copy(rows[b], acc_sh.at[dst_v.at[j]], add=True)
        if j + 2 < NB:
            pltpu.sync_copy(src4.at[cid, sid, j + 2], src_r.at[b])
            pend[b] = pltpu.async_copy(x_cat.at[src_r.at[b]], rows[b], sems[b])
    plsc.subcore_barrier()

    soff = cid * ACC_N + sid * RPT
    pltpu.sync_copy(acc_sh.at[pl.ds(sid * RPT, RPT)],
                    sum_out.at[pl.ds(soff, RPT)])


def _sc_cnt_body(dst3, ones_h, zacc, cnt_out,
                 dst_v, db, ones_v, acc_sh):
    cid = lax.axis_index("c")
    sid = lax.axis_index("s")

    zoff = sid * RPT
    pltpu.sync_copy(zacc.at[pl.ds(zoff, RPT)], acc_sh.at[pl.ds(zoff, RPT)])
    pltpu.sync_copy(ones_h, ones_v)
    pltpu.sync_copy(dst3.at[sid, pl.ds(cid * NBH, NBH)], dst_v)
    plsc.subcore_barrier()

    def step(j, carry):
        pltpu.sync_copy(ones_v, acc_sh.at[dst_v.at[j]], add=True)
        return carry

    lax.fori_loop(0, NBH, step, 0)
    plsc.subcore_barrier()

    soff = cid * ACC_N + sid * RPT
    pltpu.sync_copy(acc_sh.at[pl.ds(sid * RPT, RPT)],
                    cnt_out.at[pl.ds(soff, RPT)])


def _make_sc_sum():
    return functools.partial(
        pl.kernel,
        out_type=[jax.ShapeDtypeStruct((NC * ACC_N, HALF), jnp.float32)],
        mesh=plsc.VectorSubcoreMesh(core_axis_name="c", subcore_axis_name="s",
                                    num_cores=NC, num_subcores=NS),
        scratch_types=[
            pltpu.VMEM((2, EB), jnp.int32),
            pltpu.VMEM((NB, EB), jnp.int32),
            pltpu.VMEM((EB,), jnp.int32),
            pltpu.VMEM((EB,), jnp.int32),
            pltpu.VMEM((EB, HALF), jnp.float32),
            pltpu.VMEM((EB, HALF), jnp.float32),
            pltpu.VMEM_SHARED((ACC_N, HALF), jnp.float32),
            pltpu.SemaphoreType.DMA,
            pltpu.SemaphoreType.DMA,
        ],
    )(_sc_sum_body)


def _make_sc_cnt():
    return functools.partial(
        pl.kernel,
        out_type=[jax.ShapeDtypeStruct((NC * ACC_N, HALF), jnp.float32)],
        mesh=plsc.VectorSubcoreMesh(core_axis_name="c", subcore_axis_name="s",
                                    num_cores=NC, num_subcores=NS),
        scratch_types=[
            pltpu.VMEM((NBH, EB), jnp.int32),
            pltpu.VMEM((EB,), jnp.int32),
            pltpu.VMEM((EB, HALF), jnp.float32),
            pltpu.VMEM_SHARED((ACC_N, HALF), jnp.float32),
        ],
    )(_sc_cnt_body)


def _tc1_body(s0, s1, c0, c1, xb, wl, wr, bb, h_out, ps_out, pq_out):
    cnt = jnp.maximum(c0[:, :1] + c1[:, :1], 1.0)
    mean = jnp.concatenate([s0[...], s1[...]], axis=1) / cnt
    h = jnp.dot(mean, wl[...], preferred_element_type=jnp.float32)
    h = h + jnp.dot(xb[...], wr[...], preferred_element_type=jnp.float32)
    h = jnp.maximum(h + bb[...], 0.0)
    h_out[...] = h
    ps_out[...] = jnp.sum(h, axis=0, keepdims=True)[None]
    pq_out[...] = jnp.sum(h * h, axis=0, keepdims=True)[None]


def _tc2_body(h, ps, pq, gam, bet, y_out):
    inv_n = jnp.float32(1.0 / N_NODES)
    mu = jnp.sum(ps[...], axis=0) * inv_n
    ex2 = jnp.sum(pq[...], axis=0) * inv_n
    var = ex2 - mu * mu
    scale = lax.rsqrt(var + 1e-5) * gam[...]
    y_out[...] = (h[...] - mu) * scale + bet[...]


def kernel(x, edge_index, W_l, W_r, b, gamma, beta):
    src = edge_index[0]
    dst = edge_index[1]
    pad = E_PAD - N_EDGES
    srcp = jnp.concatenate([src, jnp.zeros((pad,), jnp.int32)])
    src4 = jnp.stack([srcp, srcp + N_NODES]).reshape(NC, NS, NB, EB)
    dst3 = jnp.concatenate(
        [dst, jnp.full((pad,), DUMMY, jnp.int32)]).reshape(NS, NB, EB)
    x_cat = jnp.concatenate([x[:, :HALF], x[:, HALF:]], axis=0)
    ones_h = jnp.ones((EB, HALF), jnp.float32)
    zacc = jnp.zeros((ACC_N, HALF), jnp.float32)

    (s,) = _make_sc_sum()(x_cat, src4, dst3, zacc)
    (c,) = _make_sc_cnt()(dst3, ones_h, zacc)
    s0 = s[:N_NODES]
    s1 = s[ACC_N:ACC_N + N_NODES]
    c0 = c[:N_NODES]
    c1 = c[ACC_N:ACC_N + N_NODES]

    h, ps, pq = pl.pallas_call(
        _tc1_body,
        grid=(G,),
        in_specs=[
            pl.BlockSpec((BR, HALF), lambda g: (g, 0)),
            pl.BlockSpec((BR, HALF), lambda g: (g, 0)),
            pl.BlockSpec((BR, HALF), lambda g: (g, 0)),
            pl.BlockSpec((BR, HALF), lambda g: (g, 0)),
            pl.BlockSpec((BR, D), lambda g: (g, 0)),
            pl.BlockSpec((D, D), lambda g: (0, 0)),
            pl.BlockSpec((D, D), lambda g: (0, 0)),
            pl.BlockSpec((1, D), lambda g: (0, 0)),
        ],
        out_specs=[
            pl.BlockSpec((BR, D), lambda g: (g, 0)),
            pl.BlockSpec((1, 1, D), lambda g: (g, 0, 0)),
            pl.BlockSpec((1, 1, D), lambda g: (g, 0, 0)),
        ],
        out_shape=[
            jax.ShapeDtypeStruct((N_NODES, D), jnp.float32),
            jax.ShapeDtypeStruct((G, 1, D), jnp.float32),
            jax.ShapeDtypeStruct((G, 1, D), jnp.float32),
        ],
    )(s0, s1, c0, c1, x, W_l.T, W_r.T, b.reshape(1, D))

    y = pl.pallas_call(
        _tc2_body,
        grid=(G,),
        in_specs=[
            pl.BlockSpec((BR, D), lambda g: (g, 0)),
            pl.BlockSpec((G, 1, D), lambda g: (0, 0, 0)),
            pl.BlockSpec((G, 1, D), lambda g: (0, 0, 0)),
            pl.BlockSpec((1, D), lambda g: (0, 0)),
            pl.BlockSpec((1, D), lambda g: (0, 0)),
        ],
        out_specs=pl.BlockSpec((BR, D), lambda g: (g, 0)),
        out_shape=jax.ShapeDtypeStruct((N_NODES, D), jnp.float32),
    )(h, ps, pq, gamma.reshape(1, D), beta.reshape(1, D))

    return y
